# Initial kernel scaffold; baseline (speedup 1.0000x reference)
import jax
import jax.numpy as jnp
from jax.experimental import pallas as pl


def kernel(feat, self_alpha, e_sim, e_cor, sim_edge_index, cor_edge_index):
    raise NotImplementedError("write your pallas kernel here")


# speedup vs baseline: 2.3830x; 2.3830x over previous
import functools

import jax
import jax.numpy as jnp
from jax import lax
from jax.experimental import pallas as pl
from jax.experimental.pallas import tpu as pltpu
from jax.experimental.pallas import tpu_sc as plsc

N_NODES = 10000
N_PAD = 10240
D = 128
DH = 64
E = 320000
NS = 16
E_T = E // NS
G = 10
K = 25
B = 80
R_T = N_PAD // NS
R_C = 128
N_RC = R_T // R_C


def _bcast_lane(v16, lane):
    idx = jnp.full((16, 1), lane, jnp.int32)
    dnums = lax.GatherDimensionNumbers(
        offset_dims=(), collapsed_slice_dims=(0,), start_index_map=(0,))
    return lax.gather(v16, idx, dnums, slice_sizes=(1,),
                      mode=lax.GatherScatterMode.PROMISE_IN_BOUNDS)


def _sc_kernel(feat_cat, alpha_pad, src_s, dst_s, w_s, src_c, dst_c, w_c,
               out2,
               acc_e, acc_i,
               eb, ib, fb, a_b, src_b, dst_b, w_b, rows_v, sem):
    c = lax.axis_index("c")
    s = lax.axis_index("s")
    zeros16 = jnp.zeros((16,), jnp.float32)

    def _zrow(r, _):
        for k in range(DH // 16):
            eb[r, pl.ds(16 * k, 16)] = zeros16
        return 0
    lax.fori_loop(0, R_C, _zrow, 0)

    base_t = s * R_T
    for q in range(N_RC):
        row0 = base_t + q * R_C
        pltpu.sync_copy(eb, acc_e.at[pl.ds(row0, R_C)])
        pltpu.sync_copy(eb, acc_i.at[pl.ds(row0, R_C)])
    plsc.subcore_barrier()

    def do_relation(src4, dst4, w4, acc):
        def group(g, _):
            pltpu.sync_copy(src4.at[s, g], src_b)
            pltpu.sync_copy(dst4.at[s, g], dst_b)
            pltpu.sync_copy(w4.at[s, g], w_b)

            coff = jnp.full((16,), c * N_PAD, jnp.int32)

            def off_row(j, _):
                for k in range(B // 16):
                    sl = pl.ds(16 * k, 16)
                    src_b[j, sl] = src_b[j, sl] + coff
                return 0
            lax.fori_loop(0, K, off_row, 0)

            def chunk(j, _):
                cp = pltpu.async_copy(feat_cat.at[src_b.at[j]], rows_v, sem)
                cp.wait()

                def grp16(t, _):
                    w16 = w_b[j, pl.ds(t * 16, 16)]
                    for lane in range(16):
                        wb = _bcast_lane(w16, lane)
                        e = t * 16 + lane
                        for k in range(DH // 16):
                            sl = pl.ds(16 * k, 16)
                            rows_v[e, sl] = rows_v[e, sl] * wb
                    return 0
                lax.fori_loop(0, B // 16, grp16, 0)
                pltpu.sync_copy(rows_v, acc.at[dst_b.at[j]], add=True)
                return 0
            lax.fori_loop(0, K, chunk, 0)
            return 0
        lax.fori_loop(0, G, group, 0)

    do_relation(src_s, dst_s, w_s, acc_e)
    do_relation(src_c, dst_c, w_c, acc_i)
    plsc.subcore_barrier()

    pltpu.sync_copy(alpha_pad.at[s], a_b)
    for q in range(N_RC):
        row0 = base_t + q * R_C
        pltpu.sync_copy(acc_e.at[pl.ds(row0, R_C)], eb)
        pltpu.sync_copy(acc_i.at[pl.ds(row0, R_C)], ib)
        pltpu.sync_copy(feat_cat.at[pl.ds(c * N_PAD + row0, R_C)], fb)

        def row16(t, _):
            a16 = a_b[pl.ds(q * R_C + t * 16, 16)]
            for lane in range(16):
                ab = _bcast_lane(a16, lane)
                r = t * 16 + lane
                for k in range(DH // 16):
                    sl = pl.ds(16 * k, 16)
                    e16 = eb[r, sl]
                    i16 = ib[r, sl]
                    f16 = fb[r, sl]
                    eb[r, sl] = jnp.maximum(e16 * i16, 0.0) + ab * f16
            return 0
        lax.fori_loop(0, R_C // 16, row16, 0)
        pltpu.sync_copy(eb, out2.at[c, pl.ds(row0, R_C)])


@jax.jit
def kernel(feat, self_alpha, e_sim, e_cor, sim_edge_index, cor_edge_index):
    feat2 = feat.reshape(N_NODES, 2, DH).transpose(1, 0, 2)
    feat_cat = jnp.zeros((2, N_PAD, DH), jnp.float32).at[:, :N_NODES].set(
        feat2).reshape(2 * N_PAD, DH)
    alpha_pad = jnp.zeros((N_PAD,), jnp.float32).at[:N_NODES].set(
        self_alpha[:, 1]).reshape(NS, R_T)
    i32 = jnp.int32
    src_s = sim_edge_index[0].astype(i32).reshape(NS, G, K, B)
    dst_s = sim_edge_index[1].astype(i32).reshape(NS, G, K, B)
    w_s = e_sim[:, 1].reshape(NS, G, K, B)
    src_c = cor_edge_index[0].astype(i32).reshape(NS, G, K, B)
    dst_c = cor_edge_index[1].astype(i32).reshape(NS, G, K, B)
    w_c = e_cor[:, 0].reshape(NS, G, K, B)

    mesh = plsc.VectorSubcoreMesh(core_axis_name="c", subcore_axis_name="s")
    out2 = pl.kernel(
        _sc_kernel,
        out_type=jax.ShapeDtypeStruct((2, N_PAD, DH), jnp.float32),
        mesh=mesh,
        compiler_params=pltpu.CompilerParams(use_tc_tiling_on_sc=False),
        scratch_types=[
            pltpu.VMEM_SHARED((N_PAD, DH), jnp.float32),
            pltpu.VMEM_SHARED((N_PAD, DH), jnp.float32),
            pltpu.VMEM((R_C, DH), jnp.float32),
            pltpu.VMEM((R_C, DH), jnp.float32),
            pltpu.VMEM((R_C, DH), jnp.float32),
            pltpu.VMEM((R_T,), jnp.float32),
            pltpu.VMEM((K, B), jnp.int32),
            pltpu.VMEM((K, B), jnp.int32),
            pltpu.VMEM((K, B), jnp.float32),
            pltpu.VMEM((B, DH), jnp.float32),
            pltpu.SemaphoreType.DMA,
        ],
    )(feat_cat, alpha_pad, src_s, dst_s, w_s, src_c, dst_c, w_c)

    return out2[:, :N_NODES].transpose(1, 0, 2).reshape(N_NODES, D)
